# Initial kernel scaffold; baseline (speedup 1.0000x reference)
import jax
import jax.numpy as jnp
from jax.experimental import pallas as pl


def kernel(embeddings, target):
    raise NotImplementedError("write your pallas kernel here")


# speedup vs baseline: 493.8211x; 493.8211x over previous
import jax
import jax.numpy as jnp
from jax.experimental import pallas as pl

MARGIN = 1.0
K_CONST = 1.1
EPS = 1e-6


def _loss_kernel(e_ref, t_ref, pos_ref, neg_ref):
    e = e_ref[...]
    t = t_ref[...]
    B = e.shape[0]
    D = e.shape[1]

    g = jax.lax.dot_general(
        e, e,
        dimension_numbers=(((1,), (1,)), ((), ())),
        preferred_element_type=jnp.float32,
        precision=jax.lax.Precision.HIGHEST,
    )
    n = jnp.sum(e * e, axis=1, keepdims=True)
    s = jnp.sum(e, axis=1, keepdims=True)

    d2 = n + n.T - 2.0 * g + (2.0 * EPS) * (s - s.T) + (D * EPS * EPS)
    d2 = jnp.maximum(d2, 0.0)
    dist = jnp.sqrt(d2)

    pos_terms = jnp.maximum(d2 - MARGIN, 0.0)
    neg_hinge = jnp.maximum(MARGIN * K_CONST - dist, 0.0)
    neg_terms = neg_hinge * neg_hinge

    row = jax.lax.broadcasted_iota(jnp.int32, (B, B), 0)
    col = jax.lax.broadcasted_iota(jnp.int32, (B, B), 1)
    upper = col > row
    same = t == t.T

    pos_ref[...] = jnp.sum(
        jnp.where(upper & same, pos_terms, 0.0)).reshape(1, 1)
    neg_ref[...] = jnp.sum(
        jnp.where(upper & (~same), neg_terms, 0.0)).reshape(1, 1)


def kernel(embeddings, target):
    B = embeddings.shape[0]
    t2d = target.astype(jnp.int32).reshape(B, 1)
    pos, neg = pl.pallas_call(
        _loss_kernel,
        out_shape=(
            jax.ShapeDtypeStruct((1, 1), jnp.float32),
            jax.ShapeDtypeStruct((1, 1), jnp.float32),
        ),
    )(embeddings.astype(jnp.float32), t2d)
    return (pos[0, 0], neg[0, 0])


# speedup vs baseline: 542.9418x; 1.0995x over previous
import jax
import jax.numpy as jnp
from jax.experimental import pallas as pl

MARGIN = 1.0
K_CONST = 1.1
EPS = 1e-6


def _loss_kernel(e_ref, t_ref, pos_ref, neg_ref):
    e = e_ref[...]
    t = t_ref[...]
    B = e.shape[0]
    D = e.shape[1]

    g = jax.lax.dot_general(
        e, e,
        dimension_numbers=(((1,), (1,)), ((), ())),
        preferred_element_type=jnp.float32,
        precision=jax.lax.Precision.DEFAULT,
    )
    n = jnp.sum(e * e, axis=1, keepdims=True)
    s = jnp.sum(e, axis=1, keepdims=True)

    d2 = n + n.T - 2.0 * g + (2.0 * EPS) * (s - s.T) + (D * EPS * EPS)
    d2 = jnp.maximum(d2, 0.0)
    dist = jnp.sqrt(d2)

    pos_terms = jnp.maximum(d2 - MARGIN, 0.0)
    neg_hinge = jnp.maximum(MARGIN * K_CONST - dist, 0.0)
    neg_terms = neg_hinge * neg_hinge

    row = jax.lax.broadcasted_iota(jnp.int32, (B, B), 0)
    col = jax.lax.broadcasted_iota(jnp.int32, (B, B), 1)
    upper = col > row
    same = t == t.T

    pos_ref[...] = jnp.sum(
        jnp.where(upper & same, pos_terms, 0.0)).reshape(1, 1)
    neg_ref[...] = jnp.sum(
        jnp.where(upper & (~same), neg_terms, 0.0)).reshape(1, 1)


def kernel(embeddings, target):
    B = embeddings.shape[0]
    t2d = target.astype(jnp.int32).reshape(B, 1)
    pos, neg = pl.pallas_call(
        _loss_kernel,
        out_shape=(
            jax.ShapeDtypeStruct((1, 1), jnp.float32),
            jax.ShapeDtypeStruct((1, 1), jnp.float32),
        ),
    )(embeddings.astype(jnp.float32), t2d)
    return (pos[0, 0], neg[0, 0])
